# Initial kernel scaffold; baseline (speedup 1.0000x reference)
import jax
import jax.numpy as jnp
from jax.experimental import pallas as pl


def kernel(x1, x2, w1, b1, w2, b2, wc1, bc1, wc2, bc2):
    raise NotImplementedError("write your pallas kernel here")


# speedup vs baseline: 1.2757x; 1.2757x over previous
import functools

import jax
import jax.numpy as jnp
from jax.experimental import pallas as pl
from jax.experimental.pallas import tpu as pltpu


def _fused_kernel(x1_ref, x2_ref,
                  w1a1_ref, w1a2_ref, w1m1_ref, w1m2_ref, b1_ref,
                  w2_ref, b2_ref,
                  wc1a_ref, wc1b_ref, bc1_ref, wc2_ref, bc2_ref,
                  o1_ref, o2_ref, *, inv_hw, lambda_c, lambda_s):
    x1 = x1_ref[0]
    x2 = x2_ref[0]
    C = x1.shape[0]

    avg1 = jnp.sum(x1, axis=1, keepdims=True) * inv_hw
    avg2 = jnp.sum(x2, axis=1, keepdims=True) * inv_hw
    max1 = jnp.max(x1, axis=1, keepdims=True)
    max2 = jnp.max(x2, axis=1, keepdims=True)
    h = (jnp.dot(w1a1_ref[...], avg1, preferred_element_type=jnp.float32)
         + jnp.dot(w1a2_ref[...], avg2, preferred_element_type=jnp.float32)
         + jnp.dot(w1m1_ref[...], max1, preferred_element_type=jnp.float32)
         + jnp.dot(w1m2_ref[...], max2, preferred_element_type=jnp.float32)
         + b1_ref[...])
    h = jnp.maximum(h, 0.0)
    z = jax.nn.sigmoid(
        jnp.dot(w2_ref[...], h, preferred_element_type=jnp.float32)
        + b2_ref[...])
    cw0 = z[0:C]
    cw1 = z[C:2 * C]

    hs = (jnp.dot(wc1a_ref[...], x1, preferred_element_type=jnp.float32)
          + jnp.dot(wc1b_ref[...], x2, preferred_element_type=jnp.float32)
          + bc1_ref[...])
    hs = jnp.maximum(hs, 0.0)
    s = jax.nn.sigmoid(
        jnp.dot(wc2_ref[...], hs, preferred_element_type=jnp.float32)
        + bc2_ref[...])
    s0 = s[0:1]
    s1 = s[1:2]

    o1_ref[0] = x1 + lambda_c * (cw1 * x2) + lambda_s * (s1 * x2)
    o2_ref[0] = x2 + lambda_c * (cw0 * x1) + lambda_s * (s0 * x1)


def kernel(x1, x2, w1, b1, w2, b2, wc1, bc1, wc2, bc2):
    B, C, H, W = x1.shape
    HW = H * W
    lambda_c = 0.5
    lambda_s = 0.5
    x1r = x1.reshape(B, C, HW)
    x2r = x2.reshape(B, C, HW)

    hid_c = w1.shape[1]
    w1a1 = w1[0 * C:1 * C, :].T
    w1a2 = w1[1 * C:2 * C, :].T
    w1m1 = w1[2 * C:3 * C, :].T
    w1m2 = w1[3 * C:4 * C, :].T
    b1c = b1.reshape(hid_c, 1)

    w2t = w2.T
    b2c = b2.reshape(2 * C, 1)

    hid_s = wc1.shape[1]
    wc1a = wc1[0:C, :].T
    wc1b = wc1[C:2 * C, :].T
    bc1c = bc1.reshape(hid_s, 1)

    wc2t = wc2.T
    bc2c = bc2.reshape(2, 1)

    img_spec = pl.BlockSpec((1, C, HW), lambda b: (b, 0, 0))

    def const2d(shape):
        return pl.BlockSpec(shape, lambda b: (0, 0))

    o1, o2 = pl.pallas_call(
        functools.partial(_fused_kernel, inv_hw=1.0 / HW,
                          lambda_c=lambda_c, lambda_s=lambda_s),
        out_shape=(jax.ShapeDtypeStruct((B, C, HW), x1.dtype),
                   jax.ShapeDtypeStruct((B, C, HW), x1.dtype)),
        grid=(B,),
        in_specs=[
            img_spec, img_spec,
            const2d((hid_c, C)), const2d((hid_c, C)),
            const2d((hid_c, C)), const2d((hid_c, C)),
            const2d((hid_c, 1)),
            const2d((2 * C, hid_c)), const2d((2 * C, 1)),
            const2d((hid_s, C)), const2d((hid_s, C)), const2d((hid_s, 1)),
            const2d((2, hid_s)), const2d((2, 1)),
        ],
        out_specs=[img_spec, img_spec],
        compiler_params=pltpu.CompilerParams(
            dimension_semantics=("parallel",)),
    )(x1r, x2r, w1a1, w1a2, w1m1, w1m2, b1c, w2t, b2c,
      wc1a, wc1b, bc1c, wc2t, bc2c)

    return o1.reshape(B, C, H, W), o2.reshape(B, C, H, W)
